# Initial kernel scaffold; baseline (speedup 1.0000x reference)
import jax
import jax.numpy as jnp
from jax.experimental import pallas as pl


def kernel(input_ids, embed_table):
    raise NotImplementedError("write your pallas kernel here")


# speedup vs baseline: 1.5307x; 1.5307x over previous
import functools

import jax
import jax.numpy as jnp
from jax import lax
from jax.experimental import pallas as pl
from jax.experimental.pallas import tpu as pltpu
from jax.experimental.pallas import tpu_sc as plsc

D_MODEL = 1024
N_TOKENS = 4 * 2048

_info = plsc.get_sparse_core_info()
NC, NS = _info.num_cores, _info.num_subcores
NW = NC * NS
B_PER_W = N_TOKENS // NW
CHUNK = 32
NCHUNK = B_PER_W // CHUNK

_mesh = plsc.VectorSubcoreMesh(core_axis_name="c", subcore_axis_name="s")


@functools.partial(
    pl.kernel,
    mesh=_mesh,
    out_type=jax.ShapeDtypeStruct((N_TOKENS, D_MODEL), jnp.float32),
    scratch_types=[
        pltpu.VMEM((NCHUNK, CHUNK), jnp.int32),
        pltpu.VMEM((2, CHUNK, D_MODEL), jnp.float32),
        pltpu.SemaphoreType.DMA,
        pltpu.SemaphoreType.DMA,
    ],
)
def _sc_gather(ids_hbm, table_hbm, out_hbm, idx_v, rows_v, sem0, sem1):
    wid = lax.axis_index("s") * NC + lax.axis_index("c")
    base = wid * B_PER_W
    sems = (sem0, sem1)
    pltpu.sync_copy(ids_hbm.at[wid], idx_v)
    copies = [None, None]
    copies[0] = pltpu.async_copy(
        table_hbm.at[idx_v.at[0]], rows_v.at[0], sems[0])
    for i in range(NCHUNK):
        if i + 1 < NCHUNK:
            copies[(i + 1) % 2] = pltpu.async_copy(
                table_hbm.at[idx_v.at[i + 1]], rows_v.at[(i + 1) % 2],
                sems[(i + 1) % 2])
        copies[i % 2].wait()
        pltpu.sync_copy(rows_v.at[i % 2],
                        out_hbm.at[pl.ds(base + i * CHUNK, CHUNK)])


def kernel(input_ids, embed_table):
    batch, seq = input_ids.shape
    ids = input_ids.astype(jnp.int32).reshape(NW, NCHUNK, CHUNK)
    out = _sc_gather(ids, embed_table)
    return out.reshape(batch, seq, D_MODEL)


# speedup vs baseline: 1.5310x; 1.0002x over previous
import functools

import jax
import jax.numpy as jnp
from jax import lax
from jax.experimental import pallas as pl
from jax.experimental.pallas import tpu as pltpu
from jax.experimental.pallas import tpu_sc as plsc

D_MODEL = 1024
N_TOKENS = 4 * 2048

_info = plsc.get_sparse_core_info()
NC, NS = _info.num_cores, _info.num_subcores
NW = NC * NS
B_PER_W = N_TOKENS // NW
CHUNK = 32
NCHUNK = B_PER_W // CHUNK

_mesh = plsc.VectorSubcoreMesh(core_axis_name="c", subcore_axis_name="s")


NBUF = 3


@functools.partial(
    pl.kernel,
    mesh=_mesh,
    out_type=jax.ShapeDtypeStruct((N_TOKENS, D_MODEL), jnp.float32),
    scratch_types=[
        pltpu.VMEM((NCHUNK, CHUNK), jnp.int32),
        pltpu.VMEM((NBUF, CHUNK, D_MODEL), jnp.float32),
    ]
    + [pltpu.SemaphoreType.DMA] * (2 * NBUF),
)
def _sc_gather(ids_hbm, table_hbm, out_hbm, idx_v, rows_v, *sems):
    gsem, ssem = sems[:NBUF], sems[NBUF:]
    wid = lax.axis_index("s") * NC + lax.axis_index("c")
    base = wid * B_PER_W
    pltpu.sync_copy(ids_hbm.at[wid], idx_v)
    g = [None] * NBUF
    s = [None] * NBUF
    g[0] = pltpu.async_copy(table_hbm.at[idx_v.at[0]], rows_v.at[0], gsem[0])
    for i in range(NCHUNK):
        b = i % NBUF
        if i + 1 < NCHUNK:
            bn = (i + 1) % NBUF
            if s[bn] is not None:
                s[bn].wait()
                s[bn] = None
            g[bn] = pltpu.async_copy(
                table_hbm.at[idx_v.at[i + 1]], rows_v.at[bn], gsem[bn])
        g[b].wait()
        s[b] = pltpu.async_copy(
            rows_v.at[b], out_hbm.at[pl.ds(base + i * CHUNK, CHUNK)], ssem[b])
    for b in range(NBUF):
        if s[b] is not None:
            s[b].wait()


def kernel(input_ids, embed_table):
    batch, seq = input_ids.shape
    ids = input_ids.astype(jnp.int32).reshape(NW, NCHUNK, CHUNK)
    out = _sc_gather(ids, embed_table)
    return out.reshape(batch, seq, D_MODEL)


# speedup vs baseline: 1.5667x; 1.0233x over previous
import functools

import jax
import jax.numpy as jnp
from jax import lax
from jax.experimental import pallas as pl
from jax.experimental.pallas import tpu as pltpu
from jax.experimental.pallas import tpu_sc as plsc

D_MODEL = 1024
BATCH = 4
SEQ = 2048
N_TOKENS = BATCH * SEQ

_info = plsc.get_sparse_core_info()
NC, NS = _info.num_cores, _info.num_subcores
NW = NC * NS
B_PER_W = N_TOKENS // NW
W_PER_ROW = SEQ // B_PER_W

CHUNKS = (56, 56, 56, 56, 32)
OFFS = (0, 56, 112, 168, 224)
MAXCH = max(CHUNKS)
NBUF = 2

_mesh = plsc.VectorSubcoreMesh(core_axis_name="c", subcore_axis_name="s")


@functools.partial(
    pl.kernel,
    mesh=_mesh,
    out_type=jax.ShapeDtypeStruct((N_TOKENS, D_MODEL), jnp.float32),
    scratch_types=[
        pltpu.VMEM((B_PER_W,), jnp.int32),
        pltpu.VMEM((NBUF, MAXCH, D_MODEL), jnp.float32),
    ]
    + [pltpu.SemaphoreType.DMA] * (2 * NBUF),
)
def _sc_gather(ids_hbm, table_hbm, out_hbm, idx_v, rows_v, *sems):
    gsem, ssem = sems[:NBUF], sems[NBUF:]
    wid = lax.axis_index("s") * NC + lax.axis_index("c")
    base = wid * B_PER_W
    row = wid // W_PER_ROW
    col = (wid % W_PER_ROW) * B_PER_W
    pltpu.sync_copy(ids_hbm.at[row, pl.ds(col, B_PER_W)], idx_v)
    n = len(CHUNKS)
    g = [None] * NBUF
    s = [None] * NBUF
    g[0] = pltpu.async_copy(
        table_hbm.at[idx_v.at[pl.ds(OFFS[0], CHUNKS[0])]],
        rows_v.at[0, pl.ds(0, CHUNKS[0])], gsem[0])
    for i in range(n):
        b = i % NBUF
        if i + 1 < n:
            bn = (i + 1) % NBUF
            if s[bn] is not None:
                s[bn].wait()
                s[bn] = None
            g[bn] = pltpu.async_copy(
                table_hbm.at[idx_v.at[pl.ds(OFFS[i + 1], CHUNKS[i + 1])]],
                rows_v.at[bn, pl.ds(0, CHUNKS[i + 1])], gsem[bn])
        g[b].wait()
        s[b] = pltpu.async_copy(
            rows_v.at[b, pl.ds(0, CHUNKS[i])],
            out_hbm.at[pl.ds(base + OFFS[i], CHUNKS[i])], ssem[b])
    for b in range(NBUF):
        if s[b] is not None:
            s[b].wait()


def kernel(input_ids, embed_table):
    out = _sc_gather(input_ids.astype(jnp.int32), embed_table)
    return out.reshape(BATCH, SEQ, D_MODEL)


# speedup vs baseline: 1.5709x; 1.0027x over previous
import functools

import jax
import jax.numpy as jnp
from jax import lax
from jax.experimental import pallas as pl
from jax.experimental.pallas import tpu as pltpu
from jax.experimental.pallas import tpu_sc as plsc

D_MODEL = 1024
BATCH = 4
SEQ = 2048
N_TOKENS = BATCH * SEQ

_info = plsc.get_sparse_core_info()
NC, NS = _info.num_cores, _info.num_subcores
NW = NC * NS
B_PER_W = N_TOKENS // NW
W_PER_ROW = SEQ // B_PER_W

CHUNKS = (24,) * 10 + (16,)
OFFS = tuple(sum(CHUNKS[:i]) for i in range(len(CHUNKS)))
MAXCH = max(CHUNKS)
NBUF = 4
AHEAD = 2

_mesh = plsc.VectorSubcoreMesh(core_axis_name="c", subcore_axis_name="s")


@functools.partial(
    pl.kernel,
    mesh=_mesh,
    out_type=jax.ShapeDtypeStruct((N_TOKENS, D_MODEL), jnp.float32),
    scratch_types=[
        pltpu.VMEM((B_PER_W,), jnp.int32),
        pltpu.VMEM((NBUF, MAXCH, D_MODEL), jnp.float32),
    ]
    + [pltpu.SemaphoreType.DMA] * (2 * NBUF),
)
def _sc_gather(ids_hbm, table_hbm, out_hbm, idx_v, rows_v, *sems):
    gsem, ssem = sems[:NBUF], sems[NBUF:]
    wid = lax.axis_index("s") * NC + lax.axis_index("c")
    base = wid * B_PER_W
    row = wid // W_PER_ROW
    col = (wid % W_PER_ROW) * B_PER_W
    pltpu.sync_copy(ids_hbm.at[row, pl.ds(col, B_PER_W)], idx_v)
    n = len(CHUNKS)
    g = [None] * NBUF
    s = [None] * NBUF

    def start_gather(j):
        b = j % NBUF
        if s[b] is not None:
            s[b].wait()
            s[b] = None
        g[b] = pltpu.async_copy(
            table_hbm.at[idx_v.at[pl.ds(OFFS[j], CHUNKS[j])]],
            rows_v.at[b, pl.ds(0, CHUNKS[j])], gsem[b])

    for j in range(min(AHEAD, n)):
        start_gather(j)
    for i in range(n):
        b = i % NBUF
        if i + AHEAD < n:
            start_gather(i + AHEAD)
        g[b].wait()
        s[b] = pltpu.async_copy(
            rows_v.at[b, pl.ds(0, CHUNKS[i])],
            out_hbm.at[pl.ds(base + OFFS[i], CHUNKS[i])], ssem[b])
    for b in range(NBUF):
        if s[b] is not None:
            s[b].wait()


def kernel(input_ids, embed_table):
    out = _sc_gather(input_ids.astype(jnp.int32), embed_table)
    return out.reshape(BATCH, SEQ, D_MODEL)
